# Initial kernel scaffold; baseline (speedup 1.0000x reference)
import jax
import jax.numpy as jnp
from jax.experimental import pallas as pl


def kernel(x, pos, edge_index, batch, W1a, b1a, W2a, b2a, W1b, b1b, W2b, b2b, Wl, bl):
    raise NotImplementedError("write your pallas kernel here")


# speedup vs baseline: 1.0070x; 1.0070x over previous
import jax
import jax.numpy as jnp
from jax.experimental import pallas as pl

_NUM_GRAPHS = 256


def _tail(mean_ref, wl_ref, bl_ref, out_ref):
    out_ref[...] = mean_ref[...] @ wl_ref[...] + bl_ref[0, 0]


def _gin_conv(h, edge_index, W1, b1, W2, b2):
    src = edge_index[0]
    dst = edge_index[1]
    agg = jnp.zeros(h.shape, h.dtype).at[dst].add(h[src])
    z = h + agg
    z = jnp.maximum(z @ W1 + b1, 0.0) @ W2 + b2
    return z


def kernel(x, pos, edge_index, batch, W1a, b1a, W2a, b2a, W1b, b1b, W2b, b2b, Wl, bl):
    h = jnp.concatenate([x, pos], axis=1)
    h = jnp.maximum(_gin_conv(h, edge_index, W1a, b1a, W2a, b2a), 0.0)
    h = jnp.maximum(_gin_conv(h, edge_index, W1b, b1b, W2b, b2b), 0.0)
    sums = jax.ops.segment_sum(h, batch, num_segments=_NUM_GRAPHS)
    counts = jax.ops.segment_sum(jnp.ones((h.shape[0], 1), h.dtype), batch,
                                 num_segments=_NUM_GRAPHS)
    mean = sums / jnp.maximum(counts, 1.0)
    out = pl.pallas_call(
        _tail,
        out_shape=jax.ShapeDtypeStruct((_NUM_GRAPHS, 1), jnp.float32),
    )(mean, Wl, bl.reshape(1, 1))
    return out


# speedup vs baseline: 13.4938x; 13.3995x over previous
import jax
import jax.numpy as jnp
from jax import lax
from jax.experimental import pallas as pl
from jax.experimental.pallas import tpu as pltpu
from jax.experimental.pallas import tpu_sc as plsc

_N = 100000
_G = 256
_H = 64
_NC, _NS = 2, 16
_NPAD = 100352
_BN = 2048
_GRID = _NPAD // _BN
_E = 3200000
_KB = 8
_EROWS = 25088
_EPAD = _EROWS * 128
_RPW_A = _EROWS // (_NC * _NS)
_NB_A = _RPW_A // _KB
_RPW_C = _EROWS // _NS
_NB_C = _RPW_C // _KB
_ZR = _NPAD // _NS

_mesh = plsc.VectorSubcoreMesh(core_axis_name="c", subcore_axis_name="s")


def _sc_batch(i, base, src_hbm, dst_hbm, tab_hbm, table, src_v, dst_v, rows_v,
              sem):
    r0 = base + i * _KB
    pltpu.sync_copy(src_hbm.at[pl.ds(r0, _KB)], src_v)
    pltpu.sync_copy(dst_hbm.at[pl.ds(r0, _KB)], dst_v)
    cps = [
        pltpu.async_copy(tab_hbm.at[src_v.at[j]],
                         rows_v.at[pl.ds(j * 128, 128)], sem)
        for j in range(_KB)
    ]
    for cp in cps:
        cp.wait()
    for j in range(_KB):
        pltpu.sync_copy(rows_v.at[pl.ds(j * 128, 128)], table.at[dst_v.at[j]],
                        add=True)


def _agg_a_body(h0, src_hbm, dst_hbm, zeros, p0, p1, table, src_v, dst_v,
                rows_v, sem):
    cid = lax.axis_index("c")
    sid = lax.axis_index("s")
    z0 = sid * _ZR
    pltpu.sync_copy(zeros.at[pl.ds(z0, _ZR)], table.at[pl.ds(z0, _ZR)])
    plsc.subcore_barrier()
    base = (cid * _NS + sid) * _RPW_A

    def body(i, carry):
        _sc_batch(i, base, src_hbm, dst_hbm, h0, table, src_v, dst_v, rows_v,
                  sem)
        return carry

    lax.fori_loop(0, _NB_A, body, 0)
    plsc.subcore_barrier()

    @pl.when(cid == 0)
    def _():
        pltpu.sync_copy(table.at[pl.ds(z0, _ZR)], p0.at[pl.ds(z0, _ZR)])

    @pl.when(cid == 1)
    def _():
        pltpu.sync_copy(table.at[pl.ds(z0, _ZR)], p1.at[pl.ds(z0, _ZR)])


def _agg_c_body(hc0, hc1, hc2, hc3, src_hbm, dst_hbm, zeros, a0, a1, a2, a3,
                table, src_v, dst_v, rows_v, sem):
    cid = lax.axis_index("c")
    sid = lax.axis_index("s")
    z0 = sid * _ZR
    base = sid * _RPW_C

    def one_pass(tab_hbm, out_hbm):
        pltpu.sync_copy(zeros.at[pl.ds(z0, _ZR)], table.at[pl.ds(z0, _ZR)])
        plsc.subcore_barrier()

        def body(i, carry):
            _sc_batch(i, base, src_hbm, dst_hbm, tab_hbm, table, src_v, dst_v,
                      rows_v, sem)
            return carry

        lax.fori_loop(0, _NB_C, body, 0)
        plsc.subcore_barrier()
        pltpu.sync_copy(table.at[pl.ds(z0, _ZR)], out_hbm.at[pl.ds(z0, _ZR)])

    @pl.when(cid == 0)
    def _():
        one_pass(hc0, a0)
        one_pass(hc1, a1)

    @pl.when(cid == 1)
    def _():
        one_pass(hc2, a2)
        one_pass(hc3, a3)


_sc_params = pltpu.CompilerParams(use_tc_tiling_on_sc=False)

_agg_a = pl.kernel(
    _agg_a_body,
    out_type=[jax.ShapeDtypeStruct((_NPAD, 16), jnp.float32)] * 2,
    mesh=_mesh,
    compiler_params=_sc_params,
    scratch_types=[
        pltpu.VMEM_SHARED((_NPAD, 16), jnp.float32),
        pltpu.VMEM((_KB, 128), jnp.int32),
        pltpu.VMEM((_KB, 128), jnp.int32),
        pltpu.VMEM((_KB * 128, 16), jnp.float32),
        pltpu.SemaphoreType.DMA,
    ],
)

_agg_c = pl.kernel(
    _agg_c_body,
    out_type=[jax.ShapeDtypeStruct((_NPAD, 16), jnp.float32)] * 4,
    mesh=_mesh,
    compiler_params=_sc_params,
    scratch_types=[
        pltpu.VMEM_SHARED((_NPAD, 16), jnp.float32),
        pltpu.VMEM((_KB, 128), jnp.int32),
        pltpu.VMEM((_KB, 128), jnp.int32),
        pltpu.VMEM((_KB * 128, 16), jnp.float32),
        pltpu.SemaphoreType.DMA,
    ],
)


def _mlp_a_body(h0, p0, p1, w1, b1, w2, b2, o0, o1, o2, o3):
    hin = h0[...] + p0[...] + p1[...]
    z = jnp.maximum(hin @ w1[...] + b1[...], 0.0)
    h1 = jnp.maximum(jnp.maximum(z @ w2[...] + b2[...], 0.0), 0.0)
    o0[...] = h1[:, 0:16]
    o1[...] = h1[:, 16:32]
    o2[...] = h1[:, 32:48]
    o3[...] = h1[:, 48:64]


def _mlp_b_body(hc0, hc1, hc2, hc3, a0, a1, a2, a3, bat, w1, b1, w2, b2, wl,
                bl, out, acc):
    i = pl.program_id(0)

    @pl.when(i == 0)
    def _():
        acc[...] = jnp.zeros_like(acc)

    hin = jnp.concatenate(
        [hc0[...] + a0[...], hc1[...] + a1[...], hc2[...] + a2[...],
         hc3[...] + a3[...]], axis=1)
    z = jnp.maximum(hin @ w1[...] + b1[...], 0.0)
    h2 = jnp.maximum(z @ w2[...] + b2[...], 0.0)
    onehot = (bat[...] == lax.broadcasted_iota(jnp.int32, (_BN, _G), 1)
              ).astype(jnp.float32)
    ext = jnp.concatenate([h2, jnp.ones((_BN, _H), jnp.float32)], axis=1)
    acc[...] += lax.dot_general(onehot, ext, (((0,), (0,)), ((), ())))

    @pl.when(i == _GRID - 1)
    def _():
        s = acc[...]
        mean = s[:, :_H] / jnp.maximum(s[:, _H:_H + 1], 1.0)
        out[...] = mean @ wl[...] + bl[0, 0]


_row_spec = pl.BlockSpec((_BN, 16), lambda i: (i, 0))


def _full(shape):
    return pl.BlockSpec(shape, lambda i: tuple(0 for _ in shape))


_mlp_a = pl.pallas_call(
    _mlp_a_body,
    grid=(_GRID,),
    in_specs=[_row_spec, _row_spec, _row_spec,
              _full((16, _H)), _full((1, _H)), _full((_H, _H)),
              _full((1, _H))],
    out_specs=[_row_spec] * 4,
    out_shape=[jax.ShapeDtypeStruct((_NPAD, 16), jnp.float32)] * 4,
)

_mlp_b = pl.pallas_call(
    _mlp_b_body,
    grid=(_GRID,),
    in_specs=[_row_spec] * 8 + [
        pl.BlockSpec((_BN, 1), lambda i: (i, 0)),
        _full((_H, _H)), _full((1, _H)), _full((_H, _H)), _full((1, _H)),
        _full((_H, 1)), _full((1, 1))],
    out_specs=_full((_G, 1)),
    out_shape=jax.ShapeDtypeStruct((_G, 1), jnp.float32),
    scratch_shapes=[pltpu.VMEM((_G, 2 * _H), jnp.float32)],
)


def kernel(x, pos, edge_index, batch, W1a, b1a, W2a, b2a, W1b, b1b, W2b, b2b,
           Wl, bl):
    h0 = jnp.concatenate([x, pos], axis=1)
    h0 = jnp.pad(h0, ((0, _NPAD - _N), (0, 16 - h0.shape[1])))
    src = jnp.pad(edge_index[0], (0, _EPAD - _E),
                  constant_values=_N).reshape(_EROWS, 128)
    dst = jnp.pad(edge_index[1], (0, _EPAD - _E),
                  constant_values=_N).reshape(_EROWS, 128)
    zeros = jnp.zeros((_NPAD, 16), jnp.float32)
    w1a = jnp.pad(W1a, ((0, 16 - W1a.shape[0]), (0, 0)))
    bat = jnp.pad(batch, (0, _NPAD - _N), constant_values=_G).reshape(_NPAD, 1)

    p0, p1 = _agg_a(h0, src, dst, zeros)
    hc = _mlp_a(h0, p0, p1, w1a, b1a.reshape(1, _H), W2a, b2a.reshape(1, _H))
    ac = _agg_c(hc[0], hc[1], hc[2], hc[3], src, dst, zeros)
    out = _mlp_b(hc[0], hc[1], hc[2], hc[3], ac[0], ac[1], ac[2], ac[3], bat,
                 W1b, b1b.reshape(1, _H), W2b, b2b.reshape(1, _H), Wl,
                 bl.reshape(1, 1))
    return out


# speedup vs baseline: 15.9555x; 1.1824x over previous
import jax
import jax.numpy as jnp
from jax import lax
from jax.experimental import pallas as pl
from jax.experimental.pallas import tpu as pltpu
from jax.experimental.pallas import tpu_sc as plsc

_N = 100000
_G = 256
_H = 64
_NC, _NS = 2, 16
_NPAD = 100352
_BN = 2048
_GRID = _NPAD // _BN
_E = 3200000
_KB = 8
_EROWS = 25088
_EPAD = _EROWS * 128
_RPW_A = _EROWS // (_NC * _NS)
_NB_A = _RPW_A // _KB
_RPW_C = _EROWS // _NS
_NB_C = _RPW_C // _KB
_ZR = _NPAD // _NS

_mesh = plsc.VectorSubcoreMesh(core_axis_name="c", subcore_axis_name="s")


def _sc_batch(i, base, src_hbm, dst_hbm, tab_hbm, table, zeros, src_v, dst_v,
              rows_v, sem_g, sem_s):
    r0 = base + i * _KB
    pltpu.sync_copy(src_hbm.at[pl.ds(r0, _KB)], src_v.at[0])
    pltpu.sync_copy(dst_hbm.at[pl.ds(r0, _KB)], dst_v.at[0])
    gcps = [
        pltpu.async_copy(tab_hbm.at[src_v.at[0, j]],
                         rows_v.at[pl.ds(j * 128, 128)], sem_g)
        for j in range(_KB)
    ]
    for j in range(_KB):
        gcps[j].wait()
        pltpu.async_copy(rows_v.at[pl.ds(j * 128, 128)],
                         table.at[dst_v.at[0, j]], sem_s, add=True)
    pltpu.make_async_copy(zeros.at[pl.ds(0, _KB * 128)], rows_v, sem_s).wait()


def _agg_a_body(h0, src_hbm, dst_hbm, zeros, p0, p1, table, src_v, dst_v,
                rows_v, sem_g, sem_s):
    cid = lax.axis_index("c")
    sid = lax.axis_index("s")
    z0 = sid * _ZR
    pltpu.sync_copy(zeros.at[pl.ds(z0, _ZR)], table.at[pl.ds(z0, _ZR)])
    plsc.subcore_barrier()
    base = (cid * _NS + sid) * _RPW_A

    def body(i, carry):
        _sc_batch(i, base, src_hbm, dst_hbm, h0, table, zeros, src_v, dst_v,
                  rows_v, sem_g, sem_s)
        return carry

    lax.fori_loop(0, _NB_A, body, 0)
    plsc.subcore_barrier()

    @pl.when(cid == 0)
    def _():
        pltpu.sync_copy(table.at[pl.ds(z0, _ZR)], p0.at[pl.ds(z0, _ZR)])

    @pl.when(cid == 1)
    def _():
        pltpu.sync_copy(table.at[pl.ds(z0, _ZR)], p1.at[pl.ds(z0, _ZR)])


def _agg_c_body(hc0, hc1, hc2, hc3, src_hbm, dst_hbm, zeros, a0, a1, a2, a3,
                table, src_v, dst_v, rows_v, sem_g, sem_s):
    cid = lax.axis_index("c")
    sid = lax.axis_index("s")
    z0 = sid * _ZR
    base = sid * _RPW_C

    def one_pass(tab_hbm, out_hbm):
        pltpu.sync_copy(zeros.at[pl.ds(z0, _ZR)], table.at[pl.ds(z0, _ZR)])
        plsc.subcore_barrier()

        def body(i, carry):
            _sc_batch(i, base, src_hbm, dst_hbm, tab_hbm, table, zeros, src_v,
                      dst_v, rows_v, sem_g, sem_s)
            return carry

        lax.fori_loop(0, _NB_C, body, 0)
        plsc.subcore_barrier()
        pltpu.sync_copy(table.at[pl.ds(z0, _ZR)], out_hbm.at[pl.ds(z0, _ZR)])

    @pl.when(cid == 0)
    def _():
        one_pass(hc0, a0)
        one_pass(hc1, a1)

    @pl.when(cid == 1)
    def _():
        one_pass(hc2, a2)
        one_pass(hc3, a3)


_sc_params = pltpu.CompilerParams(use_tc_tiling_on_sc=False)

_agg_a = pl.kernel(
    _agg_a_body,
    out_type=[jax.ShapeDtypeStruct((_NPAD, 16), jnp.float32)] * 2,
    mesh=_mesh,
    compiler_params=_sc_params,
    scratch_types=[
        pltpu.VMEM_SHARED((_NPAD, 16), jnp.float32),
        pltpu.VMEM((1, _KB, 128), jnp.int32),
        pltpu.VMEM((1, _KB, 128), jnp.int32),
        pltpu.VMEM((_KB * 128, 16), jnp.float32),
        pltpu.SemaphoreType.DMA,
        pltpu.SemaphoreType.DMA,
    ],
)

_agg_c = pl.kernel(
    _agg_c_body,
    out_type=[jax.ShapeDtypeStruct((_NPAD, 16), jnp.float32)] * 4,
    mesh=_mesh,
    compiler_params=_sc_params,
    scratch_types=[
        pltpu.VMEM_SHARED((_NPAD, 16), jnp.float32),
        pltpu.VMEM((1, _KB, 128), jnp.int32),
        pltpu.VMEM((1, _KB, 128), jnp.int32),
        pltpu.VMEM((_KB * 128, 16), jnp.float32),
        pltpu.SemaphoreType.DMA,
        pltpu.SemaphoreType.DMA,
    ],
)


def _mlp_a_body(h0, p0, p1, w1, b1, w2, b2, o0, o1, o2, o3):
    hin = h0[...] + p0[...] + p1[...]
    z = jnp.maximum(hin @ w1[...] + b1[...], 0.0)
    h1 = jnp.maximum(jnp.maximum(z @ w2[...] + b2[...], 0.0), 0.0)
    o0[...] = h1[:, 0:16]
    o1[...] = h1[:, 16:32]
    o2[...] = h1[:, 32:48]
    o3[...] = h1[:, 48:64]


def _mlp_b_body(hc0, hc1, hc2, hc3, a0, a1, a2, a3, bat, w1, b1, w2, b2, wl,
                bl, out, acc):
    i = pl.program_id(0)

    @pl.when(i == 0)
    def _():
        acc[...] = jnp.zeros_like(acc)

    hin = jnp.concatenate(
        [hc0[...] + a0[...], hc1[...] + a1[...], hc2[...] + a2[...],
         hc3[...] + a3[...]], axis=1)
    z = jnp.maximum(hin @ w1[...] + b1[...], 0.0)
    h2 = jnp.maximum(z @ w2[...] + b2[...], 0.0)
    onehot = (bat[...] == lax.broadcasted_iota(jnp.int32, (_BN, _G), 1)
              ).astype(jnp.float32)
    ext = jnp.concatenate([h2, jnp.ones((_BN, _H), jnp.float32)], axis=1)
    acc[...] += lax.dot_general(onehot, ext, (((0,), (0,)), ((), ())))

    @pl.when(i == _GRID - 1)
    def _():
        s = acc[...]
        mean = s[:, :_H] / jnp.maximum(s[:, _H:_H + 1], 1.0)
        out[...] = mean @ wl[...] + bl[0, 0]


_row_spec = pl.BlockSpec((_BN, 16), lambda i: (i, 0))


def _full(shape):
    return pl.BlockSpec(shape, lambda i: tuple(0 for _ in shape))


_mlp_a = pl.pallas_call(
    _mlp_a_body,
    grid=(_GRID,),
    in_specs=[_row_spec, _row_spec, _row_spec,
              _full((16, _H)), _full((1, _H)), _full((_H, _H)),
              _full((1, _H))],
    out_specs=[_row_spec] * 4,
    out_shape=[jax.ShapeDtypeStruct((_NPAD, 16), jnp.float32)] * 4,
)

_mlp_b = pl.pallas_call(
    _mlp_b_body,
    grid=(_GRID,),
    in_specs=[_row_spec] * 8 + [
        pl.BlockSpec((_BN, 1), lambda i: (i, 0)),
        _full((_H, _H)), _full((1, _H)), _full((_H, _H)), _full((1, _H)),
        _full((_H, 1)), _full((1, 1))],
    out_specs=_full((_G, 1)),
    out_shape=jax.ShapeDtypeStruct((_G, 1), jnp.float32),
    scratch_shapes=[pltpu.VMEM((_G, 2 * _H), jnp.float32)],
)


def kernel(x, pos, edge_index, batch, W1a, b1a, W2a, b2a, W1b, b1b, W2b, b2b,
           Wl, bl):
    h0 = jnp.concatenate([x, pos], axis=1)
    h0 = jnp.pad(h0, ((0, _NPAD - _N), (0, 16 - h0.shape[1])))
    src = jnp.pad(edge_index[0], (0, _EPAD - _E),
                  constant_values=_N).reshape(_EROWS, 128)
    dst = jnp.pad(edge_index[1], (0, _EPAD - _E),
                  constant_values=_N).reshape(_EROWS, 128)
    zeros = jnp.zeros((_NPAD, 16), jnp.float32)
    w1a = jnp.pad(W1a, ((0, 16 - W1a.shape[0]), (0, 0)))
    bat = jnp.pad(batch, (0, _NPAD - _N), constant_values=_G).reshape(_NPAD, 1)

    p0, p1 = _agg_a(h0, src, dst, zeros)
    hc = _mlp_a(h0, p0, p1, w1a, b1a.reshape(1, _H), W2a, b2a.reshape(1, _H))
    ac = _agg_c(hc[0], hc[1], hc[2], hc[3], src, dst, zeros)
    out = _mlp_b(hc[0], hc[1], hc[2], hc[3], ac[0], ac[1], ac[2], ac[3], bat,
                 W1b, b1b.reshape(1, _H), W2b, b2b.reshape(1, _H), Wl,
                 bl.reshape(1, 1))
    return out
